# baseline (speedup 1.0000x reference)
import jax
import jax.numpy as jnp
from jax import lax
from jax.experimental import pallas as pl
from jax.experimental.pallas import tpu as pltpu
from jax.experimental.pallas import tpu_sc as plsc

ITEM_N = 1000000
B = 16384
L = 50
LP = 64
H = 64
NLANE = 16
NVREG = H // NLANE
NC = 2
NS = 16
NW = NC * NS
EPW = B // NW
CHUNK = 64
NCHUNK = EPW // CHUNK
DBLK = 2048


def _sc_body(user_hbm, att_hbm, maskf_hbm, uemb, aemb, out_hbm,
             aidx_v, amask_v, arows_v, uidx_v, urows_v, sem_att, sem_stage):
    wid = lax.axis_index("s") * NC + lax.axis_index("c")
    base = wid * EPW

    def step(i, carry):
        c = i // CHUNK
        e = i % CHUNK
        cbase = base + c * CHUNK

        @pl.when(e == 0)
        def _stage():
            pltpu.sync_copy(att_hbm.at[pl.ds(cbase, CHUNK)], aidx_v)
            pltpu.sync_copy(maskf_hbm.at[pl.ds(cbase, CHUNK)], amask_v)
            pltpu.sync_copy(user_hbm.at[pl.ds(cbase, CHUNK)], uidx_v)
            pltpu.async_copy(uemb.at[uidx_v], urows_v, sem_stage).wait()
            pltpu.async_copy(aemb.at[aidx_v.at[0]],
                             arows_v.at[pl.ds(0, L)], sem_att.at[0])

        nxt = e + 1

        @pl.when(nxt < CHUNK)
        def _fire():
            pltpu.async_copy(aemb.at[aidx_v.at[nxt]],
                             arows_v.at[pl.ds((nxt % 2) * L, L)],
                             sem_att.at[nxt % 2])

        pltpu.make_async_copy(aemb.at[aidx_v.at[e]],
                              arows_v.at[pl.ds((e % 2) * L, L)],
                              sem_att.at[e % 2]).wait()

        rbase = (e % 2) * L
        accs = [urows_v[e, pl.ds(k * NLANE, NLANE)] for k in range(NVREG)]
        mvecs = [amask_v[e, pl.ds(g * NLANE, NLANE)] for g in range(LP // NLANE)]
        for l in range(L):
            m = mvecs[l // NLANE][l % NLANE]
            for k in range(NVREG):
                accs[k] = accs[k] + m * arows_v[rbase + l, pl.ds(k * NLANE, NLANE)]
        for k in range(NVREG):
            urows_v[e, pl.ds(k * NLANE, NLANE)] = accs[k]

        @pl.when(e == CHUNK - 1)
        def _flush():
            pltpu.sync_copy(urows_v, out_hbm.at[pl.ds(cbase, CHUNK)])
        return carry

    lax.fori_loop(0, EPW, step, jnp.int32(0))


def _dot_body(uvec_ref, g_ref, il_ref, out_ref):
    u = uvec_ref[...]
    d0 = jnp.sum(u * g_ref[:, 0, :], axis=1)
    d1 = jnp.sum(u * g_ref[:, 1, :], axis=1)
    out_ref[...] = jnp.where(il_ref[...] == ITEM_N - 1, d1, d0)


@jax.jit
def _run(user_list, att_list, maskf, item_list, user_emb, att_emb, item_emb):
    mesh = plsc.VectorSubcoreMesh(core_axis_name="c", subcore_axis_name="s")
    sc = pl.kernel(
        _sc_body,
        out_type=jax.ShapeDtypeStruct((B, H), jnp.float32),
        mesh=mesh,
        scratch_types=[
            pltpu.VMEM((CHUNK, L), jnp.int32),
            pltpu.VMEM((CHUNK, LP), jnp.float32),
            pltpu.VMEM((2 * L, H), jnp.float32),
            pltpu.VMEM((CHUNK,), jnp.int32),
            pltpu.VMEM((CHUNK, H), jnp.float32),
            pltpu.SemaphoreType.DMA((2,)),
            pltpu.SemaphoreType.DMA,
        ],
        compiler_params=pltpu.CompilerParams(use_tc_tiling_on_sc=False),
    )
    uvec = sc(user_list, att_list, maskf, user_emb, att_emb)

    st = jnp.minimum(item_list, ITEM_N - 2)
    gdn = lax.GatherDimensionNumbers(
        offset_dims=(1, 2), collapsed_slice_dims=(), start_index_map=(0,))
    g = lax.gather(item_emb, st[:, None], gdn, (2, H),
                   mode=lax.GatherScatterMode.PROMISE_IN_BOUNDS)

    dot = pl.pallas_call(
        _dot_body,
        grid=(B // DBLK,),
        in_specs=[
            pl.BlockSpec((DBLK, H), lambda i: (i, 0)),
            pl.BlockSpec((DBLK, 2, H), lambda i: (i, 0, 0)),
            pl.BlockSpec((DBLK,), lambda i: (i,)),
        ],
        out_specs=pl.BlockSpec((DBLK,), lambda i: (i,)),
        out_shape=jax.ShapeDtypeStruct((B,), jnp.float32),
    )
    return dot(uvec, g, item_list)


def kernel(user_list, item_list, att_list, att_mask, user_emb, item_emb, att_emb):
    maskf = jnp.pad(att_mask.astype(jnp.float32), ((0, 0), (0, LP - L)))
    return _run(user_list.astype(jnp.int32),
                att_list.astype(jnp.int32), maskf, item_list.astype(jnp.int32),
                user_emb, att_emb, item_emb)


# speedup vs baseline: 75.9645x; 75.9645x over previous
import jax
import jax.numpy as jnp
from jax import lax
from jax.experimental import pallas as pl
from jax.experimental.pallas import tpu as pltpu
from jax.experimental.pallas import tpu_sc as plsc

B = 16384
L = 50
LP = 64
H = 64
NLANE = 16
NVREG = H // NLANE
NC = 2
NS = 16
NW = NC * NS
EPW = B // NW
CHUNK = 64
NCHUNK = EPW // CHUNK
DEPTH = 4


def _body(user_hbm, att_hbm, maskf_hbm, itemv_hbm, uemb, aemb, out_hbm,
          aidx_v, amask_v, arows_v, uidx_v, urows_v, irows_v,
          scores_v, sem_att, sem_stage):
    wid = lax.axis_index("s") * NC + lax.axis_index("c")
    base = wid * EPW

    lane_iota = lax.iota(jnp.int32, NLANE)
    gdn = lax.GatherDimensionNumbers(
        offset_dims=(), collapsed_slice_dims=(0,), start_index_map=(0,))

    def lane_sum(v):
        for sh in (8, 4, 2, 1):
            perm = lax.bitwise_and(lane_iota + sh, NLANE - 1)[:, None]
            v = v + lax.gather(v, perm, gdn, (1,),
                               mode=lax.GatherScatterMode.PROMISE_IN_BOUNDS)
        return v

    def fire(e):
        pltpu.async_copy(aemb.at[aidx_v.at[e]],
                         arows_v.at[pl.ds((e % DEPTH) * L, L)],
                         sem_att.at[e % DEPTH])

    def step(i, svec):
        c = i // CHUNK
        e = i % CHUNK
        cbase = base + c * CHUNK

        @pl.when(e == 0)
        def _stage():
            pltpu.sync_copy(att_hbm.at[pl.ds(cbase, CHUNK)], aidx_v)
            pltpu.sync_copy(maskf_hbm.at[pl.ds(cbase, CHUNK)], amask_v)
            pltpu.sync_copy(user_hbm.at[pl.ds(cbase, CHUNK)], uidx_v)
            pltpu.sync_copy(itemv_hbm.at[pl.ds(cbase, CHUNK)], irows_v)
            pltpu.async_copy(uemb.at[uidx_v], urows_v, sem_stage).wait()
            for p in range(DEPTH - 1):
                fire(p)

        nxt = e + (DEPTH - 1)

        @pl.when(nxt < CHUNK)
        def _fire_ahead():
            fire(nxt)

        pltpu.make_async_copy(aemb.at[aidx_v.at[e]],
                              arows_v.at[pl.ds((e % DEPTH) * L, L)],
                              sem_att.at[e % DEPTH]).wait()

        rbase = (e % DEPTH) * L
        accs = [urows_v[e, pl.ds(k * NLANE, NLANE)] for k in range(NVREG)]
        mvecs = [amask_v[e, pl.ds(g * NLANE, NLANE)] for g in range(LP // NLANE)]
        for l in range(L):
            m = mvecs[l // NLANE][l % NLANE]
            for k in range(NVREG):
                accs[k] = accs[k] + m * arows_v[rbase + l, pl.ds(k * NLANE, NLANE)]

        tot = accs[0] * irows_v[e, pl.ds(0, NLANE)]
        for k in range(1, NVREG):
            tot = tot + accs[k] * irows_v[e, pl.ds(k * NLANE, NLANE)]
        svec = jnp.where(lane_iota == (e % NLANE), lane_sum(tot), svec)

        @pl.when((i % NLANE) == (NLANE - 1))
        def _flush():
            scores_v[pl.ds((i // NLANE) * NLANE, NLANE)] = svec
        return svec

    lax.fori_loop(0, EPW, step, jnp.zeros((NLANE,), jnp.float32))
    pltpu.sync_copy(scores_v, out_hbm.at[pl.ds(base, EPW)])


@jax.jit
def _run(user_list, att_list, maskf, item_vec, user_emb, att_emb):
    mesh = plsc.VectorSubcoreMesh(core_axis_name="c", subcore_axis_name="s")
    f = pl.kernel(
        _body,
        out_type=jax.ShapeDtypeStruct((B,), jnp.float32),
        mesh=mesh,
        scratch_types=[
            pltpu.VMEM((CHUNK, L), jnp.int32),
            pltpu.VMEM((CHUNK, LP), jnp.float32),
            pltpu.VMEM((DEPTH * L, H), jnp.float32),
            pltpu.VMEM((CHUNK,), jnp.int32),
            pltpu.VMEM((CHUNK, H), jnp.float32),
            pltpu.VMEM((CHUNK, H), jnp.float32),
            pltpu.VMEM((EPW,), jnp.float32),
            pltpu.SemaphoreType.DMA((DEPTH,)),
            pltpu.SemaphoreType.DMA,
        ],
        compiler_params=pltpu.CompilerParams(use_tc_tiling_on_sc=False),
    )
    return f(user_list, att_list, maskf, item_vec, user_emb, att_emb)


def kernel(user_list, item_list, att_list, att_mask, user_emb, item_emb, att_emb):
    maskf = jnp.pad(att_mask.astype(jnp.float32), ((0, 0), (0, LP - L)))
    item_vec = jnp.take(item_emb, item_list, axis=0)
    return _run(user_list.astype(jnp.int32),
                att_list.astype(jnp.int32), maskf, item_vec,
                user_emb, att_emb)


# speedup vs baseline: 80.6743x; 1.0620x over previous
import jax
import jax.numpy as jnp
from jax import lax
from jax.experimental import pallas as pl
from jax.experimental.pallas import tpu as pltpu
from jax.experimental.pallas import tpu_sc as plsc

B = 16384
L = 50
LP = 64
H = 64
NLANE = 16
NVREG = H // NLANE
NC = 2
NS = 16
NW = NC * NS
EPW = B // NW
CHUNK = 64
NCHUNK = EPW // CHUNK
DEPTH = 8


def _body(user_hbm, att_hbm, maskf_hbm, itemv_hbm, uemb, aemb, out_hbm,
          aidx_v, amask_v, arows_v, uidx_v, urows_v, irows_v,
          scores_v, sem_att, sem_stage):
    wid = lax.axis_index("s") * NC + lax.axis_index("c")
    base = wid * EPW

    lane_iota = lax.iota(jnp.int32, NLANE)
    gdn = lax.GatherDimensionNumbers(
        offset_dims=(), collapsed_slice_dims=(0,), start_index_map=(0,))

    def lane_sum(v):
        for sh in (8, 4, 2, 1):
            perm = lax.bitwise_and(lane_iota + sh, NLANE - 1)[:, None]
            v = v + lax.gather(v, perm, gdn, (1,),
                               mode=lax.GatherScatterMode.PROMISE_IN_BOUNDS)
        return v

    def fire(e):
        pltpu.async_copy(aemb.at[aidx_v.at[e]],
                         arows_v.at[pl.ds((e % DEPTH) * L, L)],
                         sem_att.at[e % DEPTH])

    def step(i, svec):
        c = i // CHUNK
        e = i % CHUNK
        cbase = base + c * CHUNK

        @pl.when(e == 0)
        def _stage():
            pltpu.sync_copy(att_hbm.at[pl.ds(cbase, CHUNK)], aidx_v)
            pltpu.sync_copy(maskf_hbm.at[pl.ds(cbase, CHUNK)], amask_v)
            pltpu.sync_copy(user_hbm.at[pl.ds(cbase, CHUNK)], uidx_v)
            pltpu.sync_copy(itemv_hbm.at[pl.ds(cbase, CHUNK)], irows_v)
            pltpu.async_copy(uemb.at[uidx_v], urows_v, sem_stage).wait()
            for p in range(DEPTH - 1):
                fire(p)

        nxt = e + (DEPTH - 1)

        @pl.when(nxt < CHUNK)
        def _fire_ahead():
            fire(nxt)

        pltpu.make_async_copy(aemb.at[aidx_v.at[e]],
                              arows_v.at[pl.ds((e % DEPTH) * L, L)],
                              sem_att.at[e % DEPTH]).wait()

        rbase = (e % DEPTH) * L
        accs = [urows_v[e, pl.ds(k * NLANE, NLANE)] for k in range(NVREG)]
        mvecs = [amask_v[e, pl.ds(g * NLANE, NLANE)] for g in range(LP // NLANE)]
        for l in range(L):
            m = mvecs[l // NLANE][l % NLANE]
            for k in range(NVREG):
                accs[k] = accs[k] + m * arows_v[rbase + l, pl.ds(k * NLANE, NLANE)]

        tot = accs[0] * irows_v[e, pl.ds(0, NLANE)]
        for k in range(1, NVREG):
            tot = tot + accs[k] * irows_v[e, pl.ds(k * NLANE, NLANE)]
        svec = jnp.where(lane_iota == (e % NLANE), lane_sum(tot), svec)

        @pl.when((i % NLANE) == (NLANE - 1))
        def _flush():
            scores_v[pl.ds((i // NLANE) * NLANE, NLANE)] = svec
        return svec

    lax.fori_loop(0, EPW, step, jnp.zeros((NLANE,), jnp.float32))
    pltpu.sync_copy(scores_v, out_hbm.at[pl.ds(base, EPW)])


@jax.jit
def _run(user_list, att_list, maskf, item_vec, user_emb, att_emb):
    mesh = plsc.VectorSubcoreMesh(core_axis_name="c", subcore_axis_name="s")
    f = pl.kernel(
        _body,
        out_type=jax.ShapeDtypeStruct((B,), jnp.float32),
        mesh=mesh,
        scratch_types=[
            pltpu.VMEM((CHUNK, L), jnp.int32),
            pltpu.VMEM((CHUNK, LP), jnp.float32),
            pltpu.VMEM((DEPTH * L, H), jnp.float32),
            pltpu.VMEM((CHUNK,), jnp.int32),
            pltpu.VMEM((CHUNK, H), jnp.float32),
            pltpu.VMEM((CHUNK, H), jnp.float32),
            pltpu.VMEM((EPW,), jnp.float32),
            pltpu.SemaphoreType.DMA((DEPTH,)),
            pltpu.SemaphoreType.DMA,
        ],
        compiler_params=pltpu.CompilerParams(use_tc_tiling_on_sc=False),
    )
    return f(user_list, att_list, maskf, item_vec, user_emb, att_emb)


def kernel(user_list, item_list, att_list, att_mask, user_emb, item_emb, att_emb):
    maskf = jnp.pad(att_mask.astype(jnp.float32), ((0, 0), (0, LP - L)))
    item_vec = jnp.take(item_emb, item_list, axis=0)
    return _run(user_list.astype(jnp.int32),
                att_list.astype(jnp.int32), maskf, item_vec,
                user_emb, att_emb)
